# Initial kernel scaffold; baseline (speedup 1.0000x reference)
import jax
import jax.numpy as jnp
from jax.experimental import pallas as pl


def kernel(x, shared_W, shared_b, router_W, router_b, expert_W, expert_b):
    raise NotImplementedError("write your pallas kernel here")


# speedup vs baseline: 1.8377x; 1.8377x over previous
import functools

import jax
import jax.numpy as jnp
from jax import lax
from jax.experimental import pallas as pl
from jax.experimental.pallas import tpu as pltpu

D_MODEL = 1024
NUM_EXPERTS = 8
SEQ = 2048
TOKEN_TILE = 256
NUM_TILES = SEQ // TOKEN_TILE


def _moe_body(x_ref, shared_W_ref, shared_b_ref, router_W_ref, router_b_ref,
              expert_W_ref, expert_b_ref, out_ref, acc_ref):
    e = pl.program_id(0)
    t = pl.program_id(1)

    xb = x_ref[...]

    scores = lax.dot_general(xb, router_W_ref[...],
                             (((1,), (1,)), ((), ())),
                             preferred_element_type=jnp.float32)
    scores = scores + router_b_ref[...]

    eidx = lax.broadcasted_iota(jnp.int32, scores.shape, 1)
    m0 = jnp.max(scores, axis=-1, keepdims=True)
    a0 = jnp.min(jnp.where(scores == m0, eidx, NUM_EXPERTS), axis=-1,
                 keepdims=True)
    masked = jnp.where(eidx == a0, -jnp.inf, scores)
    m1 = jnp.max(masked, axis=-1, keepdims=True)
    a1 = jnp.min(jnp.where(masked == m1, eidx, NUM_EXPERTS), axis=-1,
                 keepdims=True)
    z = jnp.exp(m1 - m0)
    w0 = 1.0 / (1.0 + z)
    w1 = z * w0
    coeff = jnp.where(a0 == e, w0, 0.0) + jnp.where(a1 == e, w1, 0.0)

    we = expert_W_ref[0]
    eo = lax.dot_general(xb, we, (((1,), (1,)), ((), ())),
                         preferred_element_type=jnp.float32)
    contrib = coeff * (eo + expert_b_ref[0])

    tsl = pl.ds(t * TOKEN_TILE, TOKEN_TILE)

    @pl.when(e == 0)
    def _():
        so = lax.dot_general(xb, shared_W_ref[...], (((1,), (1,)), ((), ())),
                             preferred_element_type=jnp.float32)
        acc_ref[tsl, :] = so + shared_b_ref[...] + contrib

    @pl.when(e > 0)
    def _():
        acc_ref[tsl, :] += contrib

    @pl.when(e == NUM_EXPERTS - 1)
    def _():
        out_ref[...] = acc_ref[tsl, :]


@jax.jit
def kernel(x, shared_W, shared_b, router_W, router_b, expert_W, expert_b):
    B, S, D = x.shape
    x2 = x.reshape(S, D)

    grid = (NUM_EXPERTS, NUM_TILES)
    out = pl.pallas_call(
        _moe_body,
        grid=grid,
        in_specs=[
            pl.BlockSpec((TOKEN_TILE, D), lambda e, t: (t, 0)),
            pl.BlockSpec((D, D), lambda e, t: (0, 0)),
            pl.BlockSpec((1, D), lambda e, t: (0, 0)),
            pl.BlockSpec((NUM_EXPERTS, D), lambda e, t: (0, 0)),
            pl.BlockSpec((1, NUM_EXPERTS), lambda e, t: (0, 0)),
            pl.BlockSpec((1, D, D), lambda e, t: (e, 0, 0)),
            pl.BlockSpec((1, 1, D), lambda e, t: (e, 0, 0)),
        ],
        out_specs=pl.BlockSpec((TOKEN_TILE, D), lambda e, t: (t, 0)),
        out_shape=jax.ShapeDtypeStruct((S, D), jnp.float32),
        scratch_shapes=[pltpu.VMEM((SEQ, D), jnp.float32)],
    )(x2, shared_W, shared_b.reshape(1, D), router_W,
      router_b.reshape(1, NUM_EXPERTS), expert_W,
      expert_b.reshape(NUM_EXPERTS, 1, D))
    return out.reshape(B, S, D)
